# baseline (device time: 114496 ns/iter reference)
import jax
import jax.numpy as jnp
from jax import lax
from jax.experimental import pallas as pl
from jax.experimental.pallas import tpu as pltpu

N_DEV = 4


def kernel(x, w_mat):
    m_per, k = x.shape
    k2, n_per = w_mat.shape
    nh = n_per // 2

    def body(x_ref, w_ref, out_hbm, xb, wb, comm_r, comm_l,
             tile_bufs, tile_rx, ostage,
             send_r, recv_r, send_l, recv_l,
             credit_r, credit_l, tile_send, tile_recv, out_sems):
        my = lax.axis_index("i")
        left = (my - 1) % N_DEV
        right = (my + 1) % N_DEV
        diag = (my + 2) % N_DEV

        wb[:, :] = w_ref[:, :].astype(jnp.bfloat16)

        barrier_sem = pltpu.get_barrier_semaphore()
        for nbr in [left, right, diag]:
            pl.semaphore_signal(
                barrier_sem, inc=1,
                device_id=(nbr,), device_id_type=pl.DeviceIdType.MESH,
            )
        pl.semaphore_wait(barrier_sem, 3)

        def mk(comm, sends, recvs, sslot, rslot, h, dst, src=None):
            return pltpu.make_async_remote_copy(
                src_ref=comm.at[sslot] if src is None else src,
                dst_ref=comm.at[rslot],
                send_sem=sends.at[h],
                recv_sem=recvs.at[h],
                device_id=(dst,),
                device_id_type=pl.DeviceIdType.MESH,
            )

        def mk_r(sslot, rslot, h, src=None):
            return mk(comm_r, send_r, recv_r, sslot, rslot, h, right, src)

        def mk_l(sslot, rslot, h, src=None):
            return mk(comm_l, send_l, recv_l, sslot, rslot, h, left, src)

        def send_tile(src, dst_rows, dst_col, dst_w, slot, dst_dev):
            t = pltpu.make_async_remote_copy(
                src_ref=src,
                dst_ref=tile_rx.at[dst_rows, :, pl.ds(dst_col, dst_w)],
                send_sem=tile_send.at[slot],
                recv_sem=tile_recv.at[slot],
                device_id=(dst_dev,),
                device_id_type=pl.DeviceIdType.MESH,
            )
            t.start()
            return t

        def flush_piece(i, rows, col, width):
            pltpu.make_async_copy(
                ostage.at[i % 2, :, pl.ds(col, width)],
                out_hbm.at[pl.ds(rows * m_per, m_per), pl.ds(col, width)],
                out_sems.at[i],
            ).start()

        def wait_flush(i, col, width):
            pltpu.make_async_copy(
                ostage.at[i % 2, :, pl.ds(col, width)],
                out_hbm.at[pl.ds(0, m_per), pl.ds(col, width)],
                out_sems.at[i],
            ).wait()

        mk_r(0, 1, 0, src=wb.at[:, pl.ds(0, nh)]).start()
        mk_l(0, 1, 0, src=wb.at[:, pl.ds(nh, nh)]).start()

        xb[:, :] = x_ref[:, :].astype(jnp.bfloat16)
        ostage[0] = jnp.dot(
            xb[:, :], wb[:, :], preferred_element_type=jnp.float32)
        flush_piece(0, my, 0, n_per)

        for h in range(N_DEV - 1):
            sslot, rslot = h % 2, (h + 1) % 2
            r = mk_r(sslot, rslot, h)
            l = mk_l(sslot, rslot, h)
            r.wait_recv()
            l.wait_recv()
            r.wait_send()
            l.wait_send()
            if h < N_DEV - 2:
                pl.semaphore_signal(credit_r, inc=1, device_id=(left,),
                                    device_id_type=pl.DeviceIdType.MESH)
                pl.semaphore_signal(credit_l, inc=1, device_id=(right,),
                                    device_id_type=pl.DeviceIdType.MESH)
                pl.semaphore_wait(credit_r, 1)
                pl.semaphore_wait(credit_l, 1)
                mk_r(rslot, sslot, h + 1).start()
                mk_l(rslot, sslot, h + 1).start()

            tile_bufs[h, :, pl.ds(0, nh)] = jnp.dot(
                xb[:, :], comm_r[rslot],
                preferred_element_type=jnp.float32).astype(jnp.bfloat16)
            tile_bufs[h, :, pl.ds(nh, nh)] = jnp.dot(
                xb[:, :], comm_l[rslot],
                preferred_element_type=jnp.float32).astype(jnp.bfloat16)
            c_r = (my - h - 1) % N_DEV
            c_l = (my + h + 1) % N_DEV
            if h == 0:
                send_tile(tile_bufs.at[0, :, pl.ds(0, nh)], 0, 0, nh, 0, c_r)
                send_tile(tile_bufs.at[0, :, pl.ds(nh, nh)], 0, nh, nh, 1, c_l)
            elif h == 1:
                send_tile(tile_bufs.at[1], 1, 0, 2 * nh, 2, c_r)
            else:
                send_tile(tile_bufs.at[2, :, pl.ds(0, nh)], 2, 0, nh, 3, c_r)
                send_tile(tile_bufs.at[2, :, pl.ds(nh, nh)], 2, nh, nh, 4, c_l)

        def wait_tile(buf, col, width, slot, src_dev):
            pltpu.make_async_remote_copy(
                src_ref=tile_bufs.at[0, :, pl.ds(0, width)],
                dst_ref=tile_rx.at[buf, :, pl.ds(col, width)],
                send_sem=tile_send.at[slot],
                recv_sem=tile_recv.at[slot],
                device_id=(src_dev,),
                device_id_type=pl.DeviceIdType.MESH,
            ).wait()

        flush_specs = {0: (my, 0, n_per)}
        drain = [
            (1, 0, 0, nh, 0, right, right),
            (2, 0, nh, nh, 1, left, left),
            (3, 1, 0, 2 * nh, 2, diag, diag),
            (4, 2, 0, nh, 3, left, left),
            (5, 2, nh, nh, 4, right, right),
        ]
        for i, buf, col, width, slot, src_dev, rows in drain:
            wait_tile(buf, col, width, slot, src_dev)
            if i >= 2:
                _, pcol, pwidth = flush_specs[i - 2]
                wait_flush(i - 2, pcol, pwidth)
            ostage[i % 2, :, pl.ds(col, width)] = (
                tile_rx[buf, :, pl.ds(col, width)].astype(jnp.float32))
            flush_piece(i, rows, col, width)
            flush_specs[i] = (rows, col, width)

        for i in (4, 5):
            _, pcol, pwidth = flush_specs[i]
            wait_flush(i, pcol, pwidth)

    return pl.pallas_call(
        body,
        out_shape=jax.ShapeDtypeStruct((N_DEV * m_per, n_per), jnp.float32),
        in_specs=[
            pl.BlockSpec(memory_space=pltpu.VMEM),
            pl.BlockSpec(memory_space=pltpu.VMEM),
        ],
        out_specs=pl.BlockSpec(memory_space=pl.ANY),
        scratch_shapes=[
            pltpu.VMEM((m_per, k), jnp.bfloat16),
            pltpu.VMEM((k, n_per), jnp.bfloat16),
            pltpu.VMEM((2, k, nh), jnp.bfloat16),
            pltpu.VMEM((2, k, nh), jnp.bfloat16),
            pltpu.VMEM((N_DEV - 1, m_per, n_per), jnp.bfloat16),
            pltpu.VMEM((N_DEV - 1, m_per, n_per), jnp.bfloat16),
            pltpu.VMEM((2, m_per, n_per), jnp.float32),
            pltpu.SemaphoreType.DMA((N_DEV - 1,)),
            pltpu.SemaphoreType.DMA((N_DEV - 1,)),
            pltpu.SemaphoreType.DMA((N_DEV - 1,)),
            pltpu.SemaphoreType.DMA((N_DEV - 1,)),
            pltpu.SemaphoreType.REGULAR,
            pltpu.SemaphoreType.REGULAR,
            pltpu.SemaphoreType.DMA((5,)),
            pltpu.SemaphoreType.DMA((5,)),
            pltpu.SemaphoreType.DMA((6,)),
        ],
        compiler_params=pltpu.CompilerParams(
            collective_id=0,
            vmem_limit_bytes=60 * 1024 * 1024,
        ),
    )(x, w_mat)


# device time: 109825 ns/iter; 1.0425x vs baseline; 1.0425x over previous
import jax
import jax.numpy as jnp
from jax import lax
from jax.experimental import pallas as pl
from jax.experimental.pallas import tpu as pltpu

N_DEV = 4


def kernel(x, w_mat):
    m_per, k = x.shape
    k2, n_per = w_mat.shape
    nh = n_per // 2

    def body(x_hbm, w_ref, out_hbm, x_stage, xb, wb, comm_r, comm_l,
             tile_bufs, tile_rx, ostage,
             send_r, recv_r, send_l, recv_l,
             credit_r, credit_l, tile_send, tile_recv, out_sems, x_sem):
        my = lax.axis_index("i")
        left = (my - 1) % N_DEV
        right = (my + 1) % N_DEV
        diag = (my + 2) % N_DEV

        cp_x = pltpu.make_async_copy(x_hbm, x_stage, x_sem)
        cp_x.start()
        wb[:, :] = w_ref[:, :].astype(jnp.bfloat16)

        barrier_sem = pltpu.get_barrier_semaphore()
        for nbr in [left, right, diag]:
            pl.semaphore_signal(
                barrier_sem, inc=1,
                device_id=(nbr,), device_id_type=pl.DeviceIdType.MESH,
            )
        pl.semaphore_wait(barrier_sem, 3)

        def mk(comm, sends, recvs, sslot, rslot, h, dst, src=None):
            return pltpu.make_async_remote_copy(
                src_ref=comm.at[sslot] if src is None else src,
                dst_ref=comm.at[rslot],
                send_sem=sends.at[h],
                recv_sem=recvs.at[h],
                device_id=(dst,),
                device_id_type=pl.DeviceIdType.MESH,
            )

        def mk_r(sslot, rslot, h, src=None):
            return mk(comm_r, send_r, recv_r, sslot, rslot, h, right, src)

        def mk_l(sslot, rslot, h, src=None):
            return mk(comm_l, send_l, recv_l, sslot, rslot, h, left, src)

        def send_tile(src, dst_rows, dst_col, dst_w, slot, dst_dev):
            t = pltpu.make_async_remote_copy(
                src_ref=src,
                dst_ref=tile_rx.at[dst_rows, :, pl.ds(dst_col, dst_w)],
                send_sem=tile_send.at[slot],
                recv_sem=tile_recv.at[slot],
                device_id=(dst_dev,),
                device_id_type=pl.DeviceIdType.MESH,
            )
            t.start()
            return t

        def flush_piece(i, rows, col, width):
            pltpu.make_async_copy(
                ostage.at[i % 2, :, pl.ds(col, width)],
                out_hbm.at[pl.ds(rows * m_per, m_per), pl.ds(col, width)],
                out_sems.at[i],
            ).start()

        def wait_flush(i, col, width):
            pltpu.make_async_copy(
                ostage.at[i % 2, :, pl.ds(col, width)],
                out_hbm.at[pl.ds(0, m_per), pl.ds(col, width)],
                out_sems.at[i],
            ).wait()

        mk_r(0, 1, 0, src=wb.at[:, pl.ds(0, nh)]).start()
        mk_l(0, 1, 0, src=wb.at[:, pl.ds(nh, nh)]).start()

        cp_x.wait()
        xb[:, :] = x_stage[:, :].astype(jnp.bfloat16)
        ostage[0] = jnp.dot(
            xb[:, :], wb[:, :], preferred_element_type=jnp.float32)
        flush_piece(0, my, 0, n_per)

        for h in range(N_DEV - 1):
            sslot, rslot = h % 2, (h + 1) % 2
            r = mk_r(sslot, rslot, h)
            l = mk_l(sslot, rslot, h)
            r.wait_recv()
            l.wait_recv()
            r.wait_send()
            l.wait_send()
            if h < N_DEV - 2:
                pl.semaphore_signal(credit_r, inc=1, device_id=(left,),
                                    device_id_type=pl.DeviceIdType.MESH)
                pl.semaphore_signal(credit_l, inc=1, device_id=(right,),
                                    device_id_type=pl.DeviceIdType.MESH)
                pl.semaphore_wait(credit_r, 1)
                pl.semaphore_wait(credit_l, 1)
                mk_r(rslot, sslot, h + 1).start()
                mk_l(rslot, sslot, h + 1).start()

            tile_bufs[h, :, pl.ds(0, nh)] = jnp.dot(
                xb[:, :], comm_r[rslot],
                preferred_element_type=jnp.float32).astype(jnp.bfloat16)
            tile_bufs[h, :, pl.ds(nh, nh)] = jnp.dot(
                xb[:, :], comm_l[rslot],
                preferred_element_type=jnp.float32).astype(jnp.bfloat16)
            c_r = (my - h - 1) % N_DEV
            c_l = (my + h + 1) % N_DEV
            if h == 0:
                send_tile(tile_bufs.at[0, :, pl.ds(0, nh)], 0, 0, nh, 0, c_r)
                send_tile(tile_bufs.at[0, :, pl.ds(nh, nh)], 0, nh, nh, 1, c_l)
            elif h == 1:
                send_tile(tile_bufs.at[1], 1, 0, 2 * nh, 2, c_r)
            else:
                send_tile(tile_bufs.at[2, :, pl.ds(0, nh)], 2, 0, nh, 3, c_r)
                send_tile(tile_bufs.at[2, :, pl.ds(nh, nh)], 2, nh, nh, 4, c_l)

        def wait_tile(buf, col, width, slot, src_dev):
            pltpu.make_async_remote_copy(
                src_ref=tile_bufs.at[0, :, pl.ds(0, width)],
                dst_ref=tile_rx.at[buf, :, pl.ds(col, width)],
                send_sem=tile_send.at[slot],
                recv_sem=tile_recv.at[slot],
                device_id=(src_dev,),
                device_id_type=pl.DeviceIdType.MESH,
            ).wait()

        flush_specs = {0: (my, 0, n_per)}
        drain = [
            (1, 0, 0, nh, 0, right, right),
            (2, 0, nh, nh, 1, left, left),
            (3, 1, 0, 2 * nh, 2, diag, diag),
            (4, 2, 0, nh, 3, left, left),
            (5, 2, nh, nh, 4, right, right),
        ]
        for i, buf, col, width, slot, src_dev, rows in drain:
            wait_tile(buf, col, width, slot, src_dev)
            if i >= 2:
                _, pcol, pwidth = flush_specs[i - 2]
                wait_flush(i - 2, pcol, pwidth)
            ostage[i % 2, :, pl.ds(col, width)] = (
                tile_rx[buf, :, pl.ds(col, width)].astype(jnp.float32))
            flush_piece(i, rows, col, width)
            flush_specs[i] = (rows, col, width)

        for i in (4, 5):
            _, pcol, pwidth = flush_specs[i]
            wait_flush(i, pcol, pwidth)

    return pl.pallas_call(
        body,
        out_shape=jax.ShapeDtypeStruct((N_DEV * m_per, n_per), jnp.float32),
        in_specs=[
            pl.BlockSpec(memory_space=pl.ANY),
            pl.BlockSpec(memory_space=pltpu.VMEM),
        ],
        out_specs=pl.BlockSpec(memory_space=pl.ANY),
        scratch_shapes=[
            pltpu.VMEM((m_per, k), jnp.float32),
            pltpu.VMEM((m_per, k), jnp.bfloat16),
            pltpu.VMEM((k, n_per), jnp.bfloat16),
            pltpu.VMEM((2, k, nh), jnp.bfloat16),
            pltpu.VMEM((2, k, nh), jnp.bfloat16),
            pltpu.VMEM((N_DEV - 1, m_per, n_per), jnp.bfloat16),
            pltpu.VMEM((N_DEV - 1, m_per, n_per), jnp.bfloat16),
            pltpu.VMEM((2, m_per, n_per), jnp.float32),
            pltpu.SemaphoreType.DMA((N_DEV - 1,)),
            pltpu.SemaphoreType.DMA((N_DEV - 1,)),
            pltpu.SemaphoreType.DMA((N_DEV - 1,)),
            pltpu.SemaphoreType.DMA((N_DEV - 1,)),
            pltpu.SemaphoreType.REGULAR,
            pltpu.SemaphoreType.REGULAR,
            pltpu.SemaphoreType.DMA((5,)),
            pltpu.SemaphoreType.DMA((5,)),
            pltpu.SemaphoreType.DMA((6,)),
            pltpu.SemaphoreType.DMA,
        ],
        compiler_params=pltpu.CompilerParams(
            collective_id=0,
            vmem_limit_bytes=60 * 1024 * 1024,
        ),
    )(x, w_mat)
